# baseline (speedup 1.0000x reference)
import functools

import jax
import jax.numpy as jnp
from jax import lax
from jax.experimental import pallas as pl
from jax.experimental.pallas import tpu as pltpu
from jax.experimental.pallas import tpu_sc as plsc

VOCAB = 100000
D = 64
B = 4096
H = 50
NB = B * H
NC = 2
NS = 16
NW = NC * NS
B_PER_W = NB // NW
R = 16
NCHUNK = (B // NW) // R
G = R * H


def _make_kernel():
    mesh = plsc.VectorSubcoreMesh(core_axis_name="c", subcore_axis_name="s")

    @functools.partial(
        pl.kernel,
        mesh=mesh,
        compiler_params=pltpu.CompilerParams(
            use_tc_tiling_on_sc=False, needs_layout_passes=False
        ),
        out_type=jax.ShapeDtypeStruct((H, D, B), jnp.float32),
        scratch_types=[
            pltpu.VMEM((B_PER_W,), jnp.int32),
            pltpu.VMEM((G, D), jnp.float32),
            pltpu.VMEM((H, D, R), jnp.float32),
            pltpu.SemaphoreType.DMA,
            pltpu.SemaphoreType.DMA,
        ],
    )
    def embed(idx_hbm, table_hbm, out_hbm, idx_v, gbuf, pbuf, gsem, wsem):
        wid = lax.axis_index("s") * NC + lax.axis_index("c")
        pltpu.sync_copy(idx_hbm.at[wid], idx_v)
        pltpu.async_copy(table_hbm.at[idx_v.at[pl.ds(0, G)]], gbuf, gsem)
        rowbase = lax.iota(jnp.int32, 16) * H

        def chunk_body(c, carry):
            pltpu.make_async_copy(
                table_hbm.at[idx_v.at[pl.ds(0, G)]], gbuf, gsem
            ).wait()

            @pl.when(c > 0)
            def _():
                pltpu.make_async_copy(pbuf, out_hbm.at[:, :, pl.ds(0, R)], wsem).wait()

            def h_body(h, hcarry):
                rows = rowbase + h
                for d in range(D):
                    dvec = jnp.full((16,), d, jnp.int32)
                    pbuf[h, d] = plsc.load_gather(gbuf, [rows, dvec])
                return hcarry

            lax.fori_loop(0, H, h_body, 0)

            pltpu.async_copy(
                pbuf, out_hbm.at[:, :, pl.ds(wid * (R * NCHUNK) + c * R, R)], wsem
            )

            @pl.when(c < NCHUNK - 1)
            def _():
                pltpu.async_copy(
                    table_hbm.at[idx_v.at[pl.ds((c + 1) * G, G)]], gbuf, gsem
                )

            return carry

        lax.fori_loop(0, NCHUNK, chunk_body, 0)
        pltpu.make_async_copy(pbuf, out_hbm.at[:, :, pl.ds(0, R)], wsem).wait()

    return embed


_embed = _make_kernel()


def kernel(batch_data, pretrained_word_embeddings):
    idx = batch_data.astype(jnp.int32).reshape(NW, B_PER_W)
    out = _embed(idx, pretrained_word_embeddings)
    return jnp.transpose(out, (2, 0, 1))


# speedup vs baseline: 1.9327x; 1.9327x over previous
import functools

import jax
import jax.numpy as jnp
from jax import lax
from jax.experimental import pallas as pl
from jax.experimental.pallas import tpu as pltpu
from jax.experimental.pallas import tpu_sc as plsc

VOCAB = 100000
D = 64
B = 4096
H = 50
NB = B * H
NC = 2
NS = 16
NW = NC * NS
B_PER_W = NB // NW
G = 640
NG = B_PER_W // G


def _make_kernel():
    mesh = plsc.VectorSubcoreMesh(core_axis_name="c", subcore_axis_name="s")
    nbuf = 2

    @functools.partial(
        pl.kernel,
        mesh=mesh,
        compiler_params=pltpu.CompilerParams(use_tc_tiling_on_sc=False),
        out_type=jax.ShapeDtypeStruct((NW, NG, G, D), jnp.float32),
        scratch_types=[
            pltpu.VMEM((B_PER_W,), jnp.int32),
            pltpu.VMEM((nbuf, G, D), jnp.float32),
        ]
        + [pltpu.SemaphoreType.DMA] * nbuf,
    )
    def embed(idx_hbm, table_hbm, out_hbm, idx_v, rows_v, *gsems):
        wid = lax.axis_index("s") * NC + lax.axis_index("c")
        pltpu.sync_copy(idx_hbm.at[wid], idx_v)

        for b in range(nbuf):
            pltpu.async_copy(
                table_hbm.at[idx_v.at[pl.ds(b * G, G)]], rows_v.at[b], gsems[b]
            )

        def body(g, carry):
            for b in range(nbuf):
                j = g * nbuf + b
                pltpu.make_async_copy(
                    table_hbm.at[idx_v.at[pl.ds(0, G)]], rows_v.at[b], gsems[b]
                ).wait()
                pltpu.sync_copy(rows_v.at[b], out_hbm.at[wid, j])
                jn = j + nbuf

                @pl.when(jn < NG)
                def _():
                    pltpu.async_copy(
                        table_hbm.at[idx_v.at[pl.ds(jn * G, G)]],
                        rows_v.at[b],
                        gsems[b],
                    )

            return carry

        lax.fori_loop(0, NG // nbuf, body, 0)

    return embed


_embed = _make_kernel()


def kernel(batch_data, pretrained_word_embeddings):
    idx = batch_data.astype(jnp.int32).reshape(NW, B_PER_W)
    out = _embed(idx, pretrained_word_embeddings)
    return out.reshape(B, H, D)
